# baseline (speedup 1.0000x reference)
import functools

import jax
import jax.numpy as jnp
from jax import lax
from jax.experimental import pallas as pl
from jax.experimental.pallas import tpu as pltpu
from jax.experimental.pallas import tpu_sc as plsc

N_NODES = 10000
N_HYPEREDGES = 10000
N_INCIDENCE = 320000
D_IN = 128
D_HIDDEN = 256

NC = 2
NS = 16
NW = NC * NS
CH = 128
NCHUNK = N_INCIDENCE // CH
CPW_LO = NCHUNK // NW
CPW_REM = NCHUNK - CPW_LO * NW
S1 = 40
S2 = CPW_LO - S1
H_PAD = 10240
ZR = H_PAD // NS


def _sc_body(x_hbm, ni_hbm, he_hbm, z_hbm, out_hbm, ni_v, he_v, rows_v0, rows_v1,
             sem0, sem1, acc_sh):
    c = lax.axis_index("c")
    s = lax.axis_index("s")
    wid = s * NC + c
    base = wid * CPW_LO + jnp.minimum(wid, CPW_REM)

    pltpu.sync_copy(z_hbm, acc_sh.at[pl.ds(s * ZR, ZR)])

    bufs = (rows_v0, rows_v1)
    sems = (sem0, sem1)
    for h, (off, cnt) in enumerate(((0, S1), (S1, S2))):
        pltpu.sync_copy(ni_hbm.at[pl.ds(base + off, cnt)], ni_v.at[pl.ds(0, cnt)])
        pltpu.sync_copy(he_hbm.at[pl.ds(base + off, cnt)], he_v.at[pl.ds(0, cnt)])
        for b in range(2):
            pltpu.async_copy(x_hbm.at[ni_v.at[b, 0]], bufs[b], sems[b])
        if h == 0:
            plsc.subcore_barrier()

        @pl.loop(0, cnt, step=2)
        def _(j):
            for b in range(2):
                jj = j + b
                pltpu.make_async_copy(
                    x_hbm.at[ni_v.at[jj, 0]], bufs[b], sems[b]).wait()
                pltpu.sync_copy(bufs[b], acc_sh.at[he_v.at[jj, 0]], add=True)

                @pl.when(jj + 2 < cnt)
                def _():
                    pltpu.async_copy(x_hbm.at[ni_v.at[jj + 2, 0]], bufs[b], sems[b])

    @pl.when(wid < CPW_REM)
    def _():
        pltpu.sync_copy(ni_hbm.at[pl.ds(base + CPW_LO, 1)], ni_v.at[pl.ds(0, 1)])
        pltpu.sync_copy(he_hbm.at[pl.ds(base + CPW_LO, 1)], he_v.at[pl.ds(0, 1)])
        pltpu.async_copy(x_hbm.at[ni_v.at[0, 0]], bufs[0], sems[0]).wait()
        pltpu.sync_copy(bufs[0], acc_sh.at[he_v.at[0, 0]], add=True)

    plsc.subcore_barrier()
    pltpu.sync_copy(acc_sh.at[pl.ds(s * ZR, ZR)],
                    out_hbm.at[c, pl.ds(s * ZR, ZR)])


_sc_seg_sum = functools.partial(
    pl.kernel,
    out_type=jax.ShapeDtypeStruct((NC, H_PAD, D_IN), jnp.float32),
    mesh=plsc.VectorSubcoreMesh(core_axis_name="c", subcore_axis_name="s"),
    scratch_types=[
        pltpu.VMEM((S1, 1, CH), jnp.int32),
        pltpu.VMEM((S1, 1, CH), jnp.int32),
        pltpu.VMEM((CH, D_IN), jnp.float32),
        pltpu.VMEM((CH, D_IN), jnp.float32),
        pltpu.SemaphoreType.DMA,
        pltpu.SemaphoreType.DMA,
        pltpu.VMEM_SHARED((H_PAD, D_IN), jnp.float32),
    ],
)(_sc_body)


_M_BLK = 1000


def _tc_node_body(x_ref, wn_ref, bn_ref, on_ref):
    on_ref[...] = jnp.maximum(
        jnp.dot(x_ref[...], wn_ref[...], preferred_element_type=jnp.float32)
        + bn_ref[...], 0.0)


def _tc_node_mlp(x, wn_t, bn):
    return pl.pallas_call(
        _tc_node_body,
        grid=(N_NODES // _M_BLK,),
        in_specs=[
            pl.BlockSpec((_M_BLK, D_IN), lambda i: (i, 0)),
            pl.BlockSpec((D_IN, D_HIDDEN), lambda i: (0, 0)),
            pl.BlockSpec((1, D_HIDDEN), lambda i: (0, 0)),
        ],
        out_specs=pl.BlockSpec((_M_BLK, D_HIDDEN), lambda i: (i, 0)),
        out_shape=jax.ShapeDtypeStruct((N_NODES, D_HIDDEN), jnp.float32),
    )(x, wn_t, bn)


def _tc_edge_body(p_ref, we_ref, be_ref, oe_ref):
    xh = p_ref[0] + p_ref[1]
    oe_ref[...] = jnp.maximum(
        jnp.dot(xh, we_ref[...], preferred_element_type=jnp.float32)
        + be_ref[...], 0.0)


def _tc_edge_mlp(partials, we_t, be):
    return pl.pallas_call(
        _tc_edge_body,
        grid=(N_HYPEREDGES // _M_BLK,),
        in_specs=[
            pl.BlockSpec((NC, _M_BLK, D_IN), lambda i: (0, i, 0)),
            pl.BlockSpec((D_IN, D_HIDDEN), lambda i: (0, 0)),
            pl.BlockSpec((1, D_HIDDEN), lambda i: (0, 0)),
        ],
        out_specs=pl.BlockSpec((_M_BLK, D_HIDDEN), lambda i: (i, 0)),
        out_shape=jax.ShapeDtypeStruct((N_HYPEREDGES, D_HIDDEN), jnp.float32),
    )(partials, we_t, be)


def kernel(x, hyperedge_index, node_index, y, batch_0, W_node, b_node, W_edge, b_edge):
    ni = node_index.astype(jnp.int32).reshape(NCHUNK, 1, CH)
    he = hyperedge_index.astype(jnp.int32).reshape(NCHUNK, 1, CH)
    zrows = jnp.zeros((ZR, D_IN), jnp.float32)
    partials = _sc_seg_sum(x, ni, he, zrows)
    out_n = _tc_node_mlp(x, W_node.T, b_node.reshape(1, D_HIDDEN))
    out_e = _tc_edge_mlp(partials, W_edge.T, b_edge.reshape(1, D_HIDDEN))
    return (y, batch_0, out_n, out_e)


# speedup vs baseline: 1.0450x; 1.0450x over previous
import functools

import jax
import jax.numpy as jnp
from jax import lax
from jax.experimental import pallas as pl
from jax.experimental.pallas import tpu as pltpu
from jax.experimental.pallas import tpu_sc as plsc

N_NODES = 10000
N_HYPEREDGES = 10000
N_INCIDENCE = 320000
D_IN = 128
D_HIDDEN = 256

NC = 2
NS = 16
NW = NC * NS
CH = 128
CPW = 80
CPH = CPW // 2
E_PAD = NW * CPW * CH
H_PAD = 10240
ZR = H_PAD // NS


def _sc_body(x_hbm, ni_hbm, he_hbm, z_hbm, out_hbm, ni_v, he_v, rows_v0, rows_v1,
             sem0, sem1, acc_sh):
    c = lax.axis_index("c")
    s = lax.axis_index("s")
    wid = s * NC + c

    pltpu.sync_copy(z_hbm, acc_sh.at[pl.ds(s * ZR, ZR)])

    bufs = (rows_v0, rows_v1)
    sems = (sem0, sem1)
    for h in range(2):
        pltpu.sync_copy(ni_hbm.at[pl.ds(wid * CPW + h * CPH, CPH)], ni_v)
        pltpu.sync_copy(he_hbm.at[pl.ds(wid * CPW + h * CPH, CPH)], he_v)
        for b in range(2):
            pltpu.async_copy(x_hbm.at[ni_v.at[b]], bufs[b], sems[b])
        if h == 0:
            plsc.subcore_barrier()

        @pl.loop(0, CPH, step=2)
        def _(j):
            for b in range(2):
                jj = j + b
                pltpu.make_async_copy(
                    x_hbm.at[ni_v.at[jj]], bufs[b], sems[b]).wait()
                pltpu.sync_copy(bufs[b], acc_sh.at[he_v.at[jj]], add=True)

                @pl.when(jj + 2 < CPH)
                def _():
                    pltpu.async_copy(x_hbm.at[ni_v.at[jj + 2]], bufs[b], sems[b])

    plsc.subcore_barrier()
    pltpu.sync_copy(acc_sh.at[pl.ds(s * ZR, ZR)],
                    out_hbm.at[c, pl.ds(s * ZR, ZR)])


_sc_seg_sum = functools.partial(
    pl.kernel,
    out_type=jax.ShapeDtypeStruct((NC, H_PAD, D_IN), jnp.float32),
    mesh=plsc.VectorSubcoreMesh(core_axis_name="c", subcore_axis_name="s"),
    scratch_types=[
        pltpu.VMEM((CPH, CH), jnp.int32),
        pltpu.VMEM((CPH, CH), jnp.int32),
        pltpu.VMEM((CH, D_IN), jnp.float32),
        pltpu.VMEM((CH, D_IN), jnp.float32),
        pltpu.SemaphoreType.DMA,
        pltpu.SemaphoreType.DMA,
        pltpu.VMEM_SHARED((H_PAD, D_IN), jnp.float32),
    ],
)(_sc_body)


_M_BLK = 1000


def _tc_node_body(x_ref, wn_ref, bn_ref, on_ref):
    on_ref[...] = jnp.maximum(
        jnp.dot(x_ref[...], wn_ref[...], preferred_element_type=jnp.float32)
        + bn_ref[...], 0.0)


def _tc_node_mlp(x, wn_t, bn):
    return pl.pallas_call(
        _tc_node_body,
        grid=(N_NODES // _M_BLK,),
        in_specs=[
            pl.BlockSpec((_M_BLK, D_IN), lambda i: (i, 0)),
            pl.BlockSpec((D_IN, D_HIDDEN), lambda i: (0, 0)),
            pl.BlockSpec((1, D_HIDDEN), lambda i: (0, 0)),
        ],
        out_specs=pl.BlockSpec((_M_BLK, D_HIDDEN), lambda i: (i, 0)),
        out_shape=jax.ShapeDtypeStruct((N_NODES, D_HIDDEN), jnp.float32),
    )(x, wn_t, bn)


def _tc_edge_body(p_ref, we_ref, be_ref, oe_ref):
    xh = p_ref[0] + p_ref[1]
    oe_ref[...] = jnp.maximum(
        jnp.dot(xh, we_ref[...], preferred_element_type=jnp.float32)
        + be_ref[...], 0.0)


def _tc_edge_mlp(partials, we_t, be):
    return pl.pallas_call(
        _tc_edge_body,
        grid=(N_HYPEREDGES // _M_BLK,),
        in_specs=[
            pl.BlockSpec((NC, _M_BLK, D_IN), lambda i: (0, i, 0)),
            pl.BlockSpec((D_IN, D_HIDDEN), lambda i: (0, 0)),
            pl.BlockSpec((1, D_HIDDEN), lambda i: (0, 0)),
        ],
        out_specs=pl.BlockSpec((_M_BLK, D_HIDDEN), lambda i: (i, 0)),
        out_shape=jax.ShapeDtypeStruct((N_HYPEREDGES, D_HIDDEN), jnp.float32),
    )(partials, we_t, be)


def kernel(x, hyperedge_index, node_index, y, batch_0, W_node, b_node, W_edge, b_edge):
    pad = E_PAD - N_INCIDENCE
    ni = jnp.concatenate(
        [node_index.astype(jnp.int32),
         jnp.arange(pad, dtype=jnp.int32) % N_NODES]
    ).reshape(NW * CPW, CH)
    he = jnp.concatenate(
        [hyperedge_index.astype(jnp.int32),
         N_HYPEREDGES + (jnp.arange(pad, dtype=jnp.int32)
                         % (H_PAD - N_HYPEREDGES))]
    ).reshape(NW * CPW, CH)
    zrows = jnp.zeros((ZR, D_IN), jnp.float32)
    partials = _sc_seg_sum(x, ni, he, zrows)
    out_n = _tc_node_mlp(x, W_node.T, b_node.reshape(1, D_HIDDEN))
    out_e = _tc_edge_mlp(partials, W_edge.T, b_edge.reshape(1, D_HIDDEN))
    return (y, batch_0, out_n, out_e)
